# Initial kernel scaffold; baseline (speedup 1.0000x reference)
import jax
import jax.numpy as jnp
from jax.experimental import pallas as pl


def kernel(x, t, s, y_t, y_s, h, active_user, length, emb, user_emb, W_ih, W_hh, b_ih, b_hh, fc_W, fc_b, trans_row, trans_col, trans_val, inter_row, inter_col, inter_val):
    raise NotImplementedError("write your pallas kernel here")


# speedup vs baseline: 3.2505x; 3.2505x over previous
import functools

import jax
import jax.numpy as jnp
import numpy as np
from jax import lax
from jax.experimental import pallas as pl
from jax.experimental.pallas import tpu as pltpu
from jax.experimental.pallas import tpu_sc as plsc

P = 10000
U = 10000
H = 128
S = 100
B = 512
E = 320000

NC = 2
NS = 16
NW = NC * NS
LANES = 16

_MESH = plsc.VectorSubcoreMesh(core_axis_name="c", subcore_axis_name="s")


_EPT = E // NW
_C = 80
_NCH = _EPT // _C
_PT = 10240
_RPT = _PT // NS


@functools.partial(
    pl.kernel,
    out_type=jax.ShapeDtypeStruct((NC, _PT, H), jnp.float32),
    mesh=_MESH,
    scratch_types=[
        pltpu.VMEM((_C,), jnp.int32),
        pltpu.VMEM((_C,), jnp.int32),
        pltpu.VMEM((_EPT,), jnp.float32),
        pltpu.VMEM((_C, H), jnp.float32),
        pltpu.VMEM_SHARED((_PT, H), jnp.float32),
        pltpu.SemaphoreType.DMA,
    ],
)
def _spmm(row_hbm, col_hbm, val_hbm, table_hbm, zero_hbm, out_hbm,
          ridx, cidx, vals, rows, acc, sem):
    cid = lax.axis_index("c")
    sid = lax.axis_index("s")
    wid = sid * NC + cid
    pltpu.sync_copy(zero_hbm.at[pl.ds(sid * _RPT, _RPT)],
                    acc.at[pl.ds(sid * _RPT, _RPT)])
    pltpu.sync_copy(val_hbm.at[pl.ds(wid * _EPT, _EPT)], vals)
    plsc.subcore_barrier()

    def chunk(i, carry):
        base = wid * _EPT + i * _C
        pltpu.sync_copy(row_hbm.at[pl.ds(base, _C)], ridx)
        pltpu.sync_copy(col_hbm.at[pl.ds(base, _C)], cidx)
        pltpu.async_copy(table_hbm.at[cidx], rows, sem).wait()

        def grpscale(g, c2):
            vv = vals[pl.ds(i * _C + g * LANES, LANES)]
            for j in range(LANES):
                bv = vv.at[jnp.full((LANES,), j, jnp.int32)].get(
                    mode="promise_in_bounds")
                rr = g * LANES + j
                for cc in range(H // LANES):
                    sl = (rr, pl.ds(cc * LANES, LANES))
                    rows[sl] = rows[sl] * bv
            return c2

        lax.fori_loop(0, _C // LANES, grpscale, 0)
        pltpu.sync_copy(rows, acc.at[ridx], add=True)
        return carry

    lax.fori_loop(0, _NCH, chunk, 0)
    plsc.subcore_barrier()
    pltpu.sync_copy(acc.at[pl.ds(sid * _RPT, _RPT)],
                    out_hbm.at[cid, pl.ds(sid * _RPT, _RPT)])


def _make_gather2(n, c):
    n_w = n // NW
    nch = n_w // c

    @functools.partial(
        pl.kernel,
        out_type=jax.ShapeDtypeStruct((n, H), jnp.float32),
        mesh=_MESH,
        scratch_types=[
            pltpu.VMEM((c,), jnp.int32),
            pltpu.VMEM((c, H), jnp.float32),
            pltpu.VMEM((c, H), jnp.float32),
            pltpu.SemaphoreType.DMA,
            pltpu.SemaphoreType.DMA,
        ],
    )
    def g2(t0_hbm, t1_hbm, idx_hbm, out_hbm, idx_v, r0, r1, s0, s1):
        cid = lax.axis_index("c")
        sid = lax.axis_index("s")
        wid = sid * NC + cid

        def chunk(i, carry):
            base = wid * n_w + i * c
            pltpu.sync_copy(idx_hbm.at[pl.ds(base, c)], idx_v)
            cp0 = pltpu.async_copy(t0_hbm.at[idx_v], r0, s0)
            cp1 = pltpu.async_copy(t1_hbm.at[idx_v], r1, s1)
            cp0.wait()
            cp1.wait()

            def rowadd(rr, c2):
                for cc in range(H // LANES):
                    sl = (rr, pl.ds(cc * LANES, LANES))
                    r0[sl] = r0[sl] + r1[sl]
                return c2

            lax.fori_loop(0, c, rowadd, 0)
            pltpu.sync_copy(r0, out_hbm.at[pl.ds(base, c)])
            return carry

        lax.fori_loop(0, nch, chunk, 0)

    return g2


_gather_x = _make_gather2(S * B, 80)


_BPT = B // NW


@functools.partial(
    pl.kernel,
    out_type=(jax.ShapeDtypeStruct((B, H), jnp.float32),
              jax.ShapeDtypeStruct((B, H), jnp.float32)),
    mesh=_MESH,
    scratch_types=[
        pltpu.VMEM((_BPT,), jnp.int32),
        pltpu.VMEM((_BPT, H), jnp.float32),
        pltpu.VMEM((_BPT, H), jnp.float32),
        pltpu.VMEM((_BPT, H), jnp.float32),
        pltpu.SemaphoreType.DMA,
        pltpu.SemaphoreType.DMA,
        pltpu.SemaphoreType.DMA,
    ],
)
def _gather_user(u0_hbm, u1_hbm, uemb_hbm, idx_hbm, up_hbm, pu_hbm,
                 idx_v, ra, rb, rc, s0, s1, s2):
    cid = lax.axis_index("c")
    sid = lax.axis_index("s")
    wid = sid * NC + cid
    base = wid * _BPT
    pltpu.sync_copy(idx_hbm.at[pl.ds(base, _BPT)], idx_v)
    cp0 = pltpu.async_copy(u0_hbm.at[idx_v], ra, s0)
    cp1 = pltpu.async_copy(u1_hbm.at[idx_v], rb, s1)
    cp2 = pltpu.async_copy(uemb_hbm.at[idx_v], rc, s2)
    cp0.wait()
    cp1.wait()
    cp2.wait()

    def rowadd(rr, c2):
        for cc in range(H // LANES):
            sl = (rr, pl.ds(cc * LANES, LANES))
            ra[sl] = ra[sl] + rb[sl]
        return c2

    lax.fori_loop(0, _BPT, rowadd, 0)
    pltpu.sync_copy(ra, up_hbm.at[pl.ds(base, _BPT)])
    pltpu.sync_copy(rc, pu_hbm.at[pl.ds(base, _BPT)])


_OMEGA = float(2.0 * np.pi / 86400.0)


def _c1_body(xe_ref, up_ref, pu_ref, t_ref, sx_ref, sy_ref, len_ref,
             h0_ref, wih_ref, whh_ref, bih_ref, bhh_ref, out_ref,
             h_s, acc_o, acc_w, tl, sxl, syl):
    i = pl.program_id(0)

    @pl.when(i == 0)
    def _init():
        h_s[...] = h0_ref[...]
        acc_o[...] = jnp.zeros_like(acc_o)
        acc_w[...] = jnp.zeros_like(acc_w)
        lm1 = len_ref[...] - 1
        ii = lax.broadcasted_iota(jnp.int32, (B, S), 1)
        selm = ii == lm1
        tl[...] = jnp.sum(jnp.where(selm, t_ref[...], 0.0), axis=1,
                          keepdims=True)
        sxl[...] = jnp.sum(jnp.where(selm, sx_ref[...], 0.0), axis=1,
                           keepdims=True)
        syl[...] = jnp.sum(jnp.where(selm, sy_ref[...], 0.0), axis=1,
                           keepdims=True)

    xe = xe_ref[0]
    d = up_ref[...] - xe
    sim = jnp.exp(-jnp.sqrt(jnp.sum(d * d, axis=1, keepdims=True) + 1e-12))

    gx = lax.dot_general(xe, wih_ref[...], (((1,), (1,)), ((), ())),
                         preferred_element_type=jnp.float32) + bih_ref[...]
    gh = lax.dot_general(h_s[...], whh_ref[...], (((1,), (1,)), ((), ())),
                         preferred_element_type=jnp.float32) + bhh_ref[...]
    xr, xz, xn = gx[:, :H], gx[:, H:2 * H], gx[:, 2 * H:]
    hr, hz, hn = gh[:, :H], gh[:, H:2 * H], gh[:, 2 * H:]
    r = jax.nn.sigmoid(xr + hr)
    z = jax.nn.sigmoid(xz + hz)
    nn = jnp.tanh(xn + r * hn)
    h_new = (1.0 - z) * nn + z * h_s[...]
    valid = i < len_ref[...]
    h_s[...] = jnp.where(valid, h_new, h_s[...])
    o = jnp.where(valid, h_new, 0.0)

    ii = lax.broadcasted_iota(jnp.int32, (B, S), 1)
    sel = ii == i
    t_i = jnp.sum(jnp.where(sel, t_ref[...], 0.0), axis=1, keepdims=True)
    sx_i = jnp.sum(jnp.where(sel, sx_ref[...], 0.0), axis=1, keepdims=True)
    sy_i = jnp.sum(jnp.where(sel, sy_ref[...], 0.0), axis=1, keepdims=True)

    dt = tl[...] - t_i
    a = (jnp.cos(dt * _OMEGA) + 1.0) * 0.5 * jnp.exp(dt * (-1e-5))
    dsx = sxl[...] - sx_i
    dsy = syl[...] - sy_i
    bw = jnp.exp(-jnp.sqrt(dsx * dsx + dsy * dsy + 1e-12))
    w = a * bw * sim
    w = jnp.where(valid, w, 0.0)
    acc_o[...] = acc_o[...] + w * o
    acc_w[...] = acc_w[...] + w

    @pl.when(i == S - 1)
    def _fin():
        out_ref[:, :H] = acc_o[...] / acc_w[...]
        out_ref[:, H:] = pu_ref[...]


def _c1(x_emb3, up, pu, t_bt, sx_bt, sy_bt, len_b1, h0,
        W_ih, W_hh, b_ih2, b_hh2):
    full = lambda shape: pl.BlockSpec(shape, lambda i: tuple(0 for _ in shape))
    return pl.pallas_call(
        _c1_body,
        grid=(S,),
        in_specs=[
            pl.BlockSpec((1, B, H), lambda i: (i, 0, 0)),
            full((B, H)), full((B, H)),
            full((B, S)), full((B, S)), full((B, S)),
            full((B, 1)), full((B, H)),
            full((3 * H, H)), full((3 * H, H)),
            full((1, 3 * H)), full((1, 3 * H)),
        ],
        out_specs=full((B, 2 * H)),
        out_shape=jax.ShapeDtypeStruct((B, 2 * H), jnp.float32),
        scratch_shapes=[
            pltpu.VMEM((B, H), jnp.float32),
            pltpu.VMEM((B, H), jnp.float32),
            pltpu.VMEM((B, 1), jnp.float32),
            pltpu.VMEM((B, 1), jnp.float32),
            pltpu.VMEM((B, 1), jnp.float32),
            pltpu.VMEM((B, 1), jnp.float32),
        ],
    )(x_emb3, up, pu, t_bt, sx_bt, sy_bt, len_b1, h0, W_ih, W_hh,
      b_ih2, b_hh2)


_PPAD = 10240
_CP = 2048


def _c2_body(pu_ref, w_ref, b_ref, out_ref):
    out_ref[...] = lax.dot_general(
        pu_ref[...], w_ref[...], (((1,), (1,)), ((), ())),
        preferred_element_type=jnp.float32) + b_ref[...]


def _c2(out_pu, fc_Wp, fc_b2p):
    return pl.pallas_call(
        _c2_body,
        grid=(_PPAD // _CP,),
        in_specs=[
            pl.BlockSpec((B, 2 * H), lambda i: (0, 0)),
            pl.BlockSpec((_CP, 2 * H), lambda i: (i, 0)),
            pl.BlockSpec((1, _CP), lambda i: (0, i)),
        ],
        out_specs=pl.BlockSpec((B, _CP), lambda i: (0, i)),
        out_shape=jax.ShapeDtypeStruct((B, _PPAD), jnp.float32),
    )(out_pu, fc_Wp, fc_b2p)


def kernel(x, t, s, y_t, y_s, h, active_user, length, emb, user_emb,
           W_ih, W_hh, b_ih, b_hh, fc_W, fc_b,
           trans_row, trans_col, trans_val, inter_row, inter_col, inter_val):
    x_flat = x.reshape(-1).astype(jnp.int32)
    au = active_user.reshape(-1).astype(jnp.int32)
    zeros_tab = jnp.zeros((_PT, H), jnp.float32)

    tp = _spmm(trans_row.astype(jnp.int32), trans_col.astype(jnp.int32),
               trans_val, emb, zeros_tab)
    upar = _spmm(inter_row.astype(jnp.int32), inter_col.astype(jnp.int32),
                 inter_val, emb, zeros_tab)

    x_emb = _gather_x(tp[0], tp[1], x_flat)
    up, pu = _gather_user(upar[0], upar[1], user_emb, au)

    out_pu = _c1(x_emb.reshape(S, B, H), up, pu,
                 t.T, s[..., 0].T, s[..., 1].T,
                 length.reshape(B, 1).astype(jnp.int32), h[0],
                 W_ih, W_hh, b_ih.reshape(1, -1), b_hh.reshape(1, -1))
    fc_Wp = jnp.pad(fc_W, ((0, _PPAD - P), (0, 0)))
    fc_b2p = jnp.pad(fc_b, (0, _PPAD - P)).reshape(1, -1)
    return _c2(out_pu, fc_Wp, fc_b2p)[:, :P]


# speedup vs baseline: 3.3854x; 1.0415x over previous
import functools

import jax
import jax.numpy as jnp
import numpy as np
from jax import lax
from jax.experimental import pallas as pl
from jax.experimental.pallas import tpu as pltpu
from jax.experimental.pallas import tpu_sc as plsc

P = 10000
U = 10000
H = 128
S = 100
B = 512
E = 320000

NC = 2
NS = 16
NW = NC * NS
LANES = 16

_MESH = plsc.VectorSubcoreMesh(core_axis_name="c", subcore_axis_name="s")


_EPT = E // NW
_C = 80
_NCH = _EPT // _C
_PT = 10240
_RPT = _PT // NS


@functools.partial(
    pl.kernel,
    out_type=jax.ShapeDtypeStruct((NC, _PT, H), jnp.float32),
    mesh=_MESH,
    scratch_types=[
        pltpu.VMEM((_C,), jnp.int32),
        pltpu.VMEM((_C,), jnp.int32),
        pltpu.VMEM((_EPT,), jnp.float32),
        pltpu.VMEM((_C, H), jnp.float32),
        pltpu.VMEM_SHARED((_PT, H), jnp.float32),
        pltpu.SemaphoreType.DMA,
    ],
)
def _spmm(row_hbm, col_hbm, val_hbm, table_hbm, out_hbm,
          ridx, cidx, vals, rows, acc, sem):
    cid = lax.axis_index("c")
    sid = lax.axis_index("s")
    wid = sid * NC + cid
    zv = jnp.zeros((LANES,), jnp.float32)

    def zrow(rr, c2):
        for cc in range(H // LANES):
            rows[rr, pl.ds(cc * LANES, LANES)] = zv
        return c2

    lax.fori_loop(0, _C, zrow, 0)
    for k in range(_RPT // _C):
        pltpu.sync_copy(rows, acc.at[pl.ds(sid * _RPT + k * _C, _C)])
    pltpu.sync_copy(val_hbm.at[pl.ds(wid * _EPT, _EPT)], vals)
    plsc.subcore_barrier()

    def chunk(i, carry):
        base = wid * _EPT + i * _C
        pltpu.sync_copy(row_hbm.at[pl.ds(base, _C)], ridx)
        pltpu.sync_copy(col_hbm.at[pl.ds(base, _C)], cidx)
        pltpu.async_copy(table_hbm.at[cidx], rows, sem).wait()

        def grpscale(g, c2):
            vv = vals[pl.ds(i * _C + g * LANES, LANES)]
            for j in range(LANES):
                bv = vv.at[jnp.full((LANES,), j, jnp.int32)].get(
                    mode="promise_in_bounds")
                rr = g * LANES + j
                for cc in range(H // LANES):
                    sl = (rr, pl.ds(cc * LANES, LANES))
                    rows[sl] = rows[sl] * bv
            return c2

        lax.fori_loop(0, _C // LANES, grpscale, 0)
        pltpu.sync_copy(rows, acc.at[ridx], add=True)
        return carry

    lax.fori_loop(0, _NCH, chunk, 0)
    plsc.subcore_barrier()
    pltpu.sync_copy(acc.at[pl.ds(sid * _RPT, _RPT)],
                    out_hbm.at[cid, pl.ds(sid * _RPT, _RPT)])


_ABLK = 2560


def _addtab_body(in_ref, out_ref):
    out_ref[...] = in_ref[0] + in_ref[1]


def _addtab(parts):
    return pl.pallas_call(
        _addtab_body,
        grid=(_PT // _ABLK,),
        in_specs=[pl.BlockSpec((2, _ABLK, H), lambda i: (0, i, 0))],
        out_specs=pl.BlockSpec((_ABLK, H), lambda i: (i, 0)),
        out_shape=jax.ShapeDtypeStruct((_PT, H), jnp.float32),
    )(parts)


def _make_gather1(n, c):
    n_w = n // NW
    nch = n_w // c

    @functools.partial(
        pl.kernel,
        out_type=jax.ShapeDtypeStruct((n, H), jnp.float32),
        mesh=_MESH,
        scratch_types=[
            pltpu.VMEM((c,), jnp.int32),
            pltpu.VMEM((c,), jnp.int32),
            pltpu.VMEM((c, H), jnp.float32),
            pltpu.VMEM((c, H), jnp.float32),
            pltpu.SemaphoreType.DMA,
            pltpu.SemaphoreType.DMA,
        ],
    )
    def g1(tab_hbm, idx_hbm, out_hbm, idx0, idx1, r0, r1, s0, s1):
        cid = lax.axis_index("c")
        sid = lax.axis_index("s")
        wid = sid * NC + cid
        base0 = wid * n_w
        pltpu.sync_copy(idx_hbm.at[pl.ds(base0, c)], idx0)
        pltpu.async_copy(tab_hbm.at[idx0], r0, s0)

        def chunk(i2, carry):
            i = i2 * 2
            pltpu.sync_copy(idx_hbm.at[pl.ds(base0 + (i + 1) * c, c)], idx1)
            pltpu.async_copy(tab_hbm.at[idx1], r1, s1)
            pltpu.make_async_copy(tab_hbm.at[idx0], r0, s0).wait()
            pltpu.sync_copy(r0, out_hbm.at[pl.ds(base0 + i * c, c)])

            @pl.when(i2 < nch // 2 - 1)
            def _pref():
                pltpu.sync_copy(idx_hbm.at[pl.ds(base0 + (i + 2) * c, c)],
                                idx0)
                pltpu.async_copy(tab_hbm.at[idx0], r0, s0)

            pltpu.make_async_copy(tab_hbm.at[idx1], r1, s1).wait()
            pltpu.sync_copy(r1, out_hbm.at[pl.ds(base0 + (i + 1) * c, c)])
            return carry

        lax.fori_loop(0, nch // 2, chunk, 0)

    return g1


_gather_x = _make_gather1(S * B, 80)


_BPT = B // NW


@functools.partial(
    pl.kernel,
    out_type=(jax.ShapeDtypeStruct((B, H), jnp.float32),
              jax.ShapeDtypeStruct((B, H), jnp.float32)),
    mesh=_MESH,
    scratch_types=[
        pltpu.VMEM((_BPT,), jnp.int32),
        pltpu.VMEM((_BPT, H), jnp.float32),
        pltpu.VMEM((_BPT, H), jnp.float32),
        pltpu.VMEM((_BPT, H), jnp.float32),
        pltpu.SemaphoreType.DMA,
        pltpu.SemaphoreType.DMA,
        pltpu.SemaphoreType.DMA,
    ],
)
def _gather_user(u0_hbm, u1_hbm, uemb_hbm, idx_hbm, up_hbm, pu_hbm,
                 idx_v, ra, rb, rc, s0, s1, s2):
    cid = lax.axis_index("c")
    sid = lax.axis_index("s")
    wid = sid * NC + cid
    base = wid * _BPT
    pltpu.sync_copy(idx_hbm.at[pl.ds(base, _BPT)], idx_v)
    cp0 = pltpu.async_copy(u0_hbm.at[idx_v], ra, s0)
    cp1 = pltpu.async_copy(u1_hbm.at[idx_v], rb, s1)
    cp2 = pltpu.async_copy(uemb_hbm.at[idx_v], rc, s2)
    cp0.wait()
    cp1.wait()
    cp2.wait()

    def rowadd(rr, c2):
        for cc in range(H // LANES):
            sl = (rr, pl.ds(cc * LANES, LANES))
            ra[sl] = ra[sl] + rb[sl]
        return c2

    lax.fori_loop(0, _BPT, rowadd, 0)
    pltpu.sync_copy(ra, up_hbm.at[pl.ds(base, _BPT)])
    pltpu.sync_copy(rc, pu_hbm.at[pl.ds(base, _BPT)])


_OMEGA = float(2.0 * np.pi / 86400.0)


def _c1_body(xe_ref, up_ref, pu_ref, t_ref, sx_ref, sy_ref, len_ref,
             h0_ref, wih_ref, whh_ref, bih_ref, bhh_ref, out_ref,
             h_s, acc_o, acc_w, tl, sxl, syl):
    i = pl.program_id(0)

    @pl.when(i == 0)
    def _init():
        h_s[...] = h0_ref[...]
        acc_o[...] = jnp.zeros_like(acc_o)
        acc_w[...] = jnp.zeros_like(acc_w)
        lm1 = len_ref[...] - 1
        ii = lax.broadcasted_iota(jnp.int32, (B, S), 1)
        selm = ii == lm1
        tl[...] = jnp.sum(jnp.where(selm, t_ref[...], 0.0), axis=1,
                          keepdims=True)
        sxl[...] = jnp.sum(jnp.where(selm, sx_ref[...], 0.0), axis=1,
                           keepdims=True)
        syl[...] = jnp.sum(jnp.where(selm, sy_ref[...], 0.0), axis=1,
                           keepdims=True)

    xe = xe_ref[0]
    d = up_ref[...] - xe
    sim = jnp.exp(-jnp.sqrt(jnp.sum(d * d, axis=1, keepdims=True) + 1e-12))

    gx = lax.dot_general(xe, wih_ref[...], (((1,), (1,)), ((), ())),
                         preferred_element_type=jnp.float32) + bih_ref[...]
    gh = lax.dot_general(h_s[...], whh_ref[...], (((1,), (1,)), ((), ())),
                         preferred_element_type=jnp.float32) + bhh_ref[...]
    xr, xz, xn = gx[:, :H], gx[:, H:2 * H], gx[:, 2 * H:]
    hr, hz, hn = gh[:, :H], gh[:, H:2 * H], gh[:, 2 * H:]
    r = jax.nn.sigmoid(xr + hr)
    z = jax.nn.sigmoid(xz + hz)
    nn = jnp.tanh(xn + r * hn)
    h_new = (1.0 - z) * nn + z * h_s[...]
    valid = i < len_ref[...]
    h_s[...] = jnp.where(valid, h_new, h_s[...])
    o = jnp.where(valid, h_new, 0.0)

    ii = lax.broadcasted_iota(jnp.int32, (B, S), 1)
    sel = ii == i
    t_i = jnp.sum(jnp.where(sel, t_ref[...], 0.0), axis=1, keepdims=True)
    sx_i = jnp.sum(jnp.where(sel, sx_ref[...], 0.0), axis=1, keepdims=True)
    sy_i = jnp.sum(jnp.where(sel, sy_ref[...], 0.0), axis=1, keepdims=True)

    dt = tl[...] - t_i
    a = (jnp.cos(dt * _OMEGA) + 1.0) * 0.5 * jnp.exp(dt * (-1e-5))
    dsx = sxl[...] - sx_i
    dsy = syl[...] - sy_i
    bw = jnp.exp(-jnp.sqrt(dsx * dsx + dsy * dsy + 1e-12))
    w = a * bw * sim
    w = jnp.where(valid, w, 0.0)
    acc_o[...] = acc_o[...] + w * o
    acc_w[...] = acc_w[...] + w

    @pl.when(i == S - 1)
    def _fin():
        out_ref[:, :H] = acc_o[...] / acc_w[...]
        out_ref[:, H:] = pu_ref[...]


def _c1(x_emb3, up, pu, t_bt, sx_bt, sy_bt, len_b1, h0,
        W_ih, W_hh, b_ih2, b_hh2):
    full = lambda shape: pl.BlockSpec(shape, lambda i: tuple(0 for _ in shape))
    return pl.pallas_call(
        _c1_body,
        grid=(S,),
        in_specs=[
            pl.BlockSpec((1, B, H), lambda i: (i, 0, 0)),
            full((B, H)), full((B, H)),
            full((B, S)), full((B, S)), full((B, S)),
            full((B, 1)), full((B, H)),
            full((3 * H, H)), full((3 * H, H)),
            full((1, 3 * H)), full((1, 3 * H)),
        ],
        out_specs=full((B, 2 * H)),
        out_shape=jax.ShapeDtypeStruct((B, 2 * H), jnp.float32),
        scratch_shapes=[
            pltpu.VMEM((B, H), jnp.float32),
            pltpu.VMEM((B, H), jnp.float32),
            pltpu.VMEM((B, 1), jnp.float32),
            pltpu.VMEM((B, 1), jnp.float32),
            pltpu.VMEM((B, 1), jnp.float32),
            pltpu.VMEM((B, 1), jnp.float32),
        ],
    )(x_emb3, up, pu, t_bt, sx_bt, sy_bt, len_b1, h0, W_ih, W_hh,
      b_ih2, b_hh2)


_PPAD = 10240
_CP = 2048


def _c2_body(pu_ref, w_ref, b_ref, out_ref):
    out_ref[...] = lax.dot_general(
        pu_ref[...], w_ref[...], (((1,), (1,)), ((), ())),
        preferred_element_type=jnp.float32) + b_ref[...]


def _c2(out_pu, fc_Wp, fc_b2p):
    return pl.pallas_call(
        _c2_body,
        grid=(_PPAD // _CP,),
        in_specs=[
            pl.BlockSpec((B, 2 * H), lambda i: (0, 0)),
            pl.BlockSpec((_CP, 2 * H), lambda i: (i, 0)),
            pl.BlockSpec((1, _CP), lambda i: (0, i)),
        ],
        out_specs=pl.BlockSpec((B, _CP), lambda i: (0, i)),
        out_shape=jax.ShapeDtypeStruct((B, _PPAD), jnp.float32),
    )(out_pu, fc_Wp, fc_b2p)


def kernel(x, t, s, y_t, y_s, h, active_user, length, emb, user_emb,
           W_ih, W_hh, b_ih, b_hh, fc_W, fc_b,
           trans_row, trans_col, trans_val, inter_row, inter_col, inter_val):
    x_flat = x.reshape(-1).astype(jnp.int32)
    au = active_user.reshape(-1).astype(jnp.int32)

    tp = _spmm(trans_row.astype(jnp.int32), trans_col.astype(jnp.int32),
               trans_val, emb)
    upar = _spmm(inter_row.astype(jnp.int32), inter_col.astype(jnp.int32),
                 inter_val, emb)

    x_emb = _gather_x(_addtab(tp), x_flat)
    up, pu = _gather_user(upar[0], upar[1], user_emb, au)

    out_pu = _c1(x_emb.reshape(S, B, H), up, pu,
                 t.T, s[..., 0].T, s[..., 1].T,
                 length.reshape(B, 1).astype(jnp.int32), h[0],
                 W_ih, W_hh, b_ih.reshape(1, -1), b_hh.reshape(1, -1))
    fc_Wp = jnp.pad(fc_W, ((0, _PPAD - P), (0, 0)))
    fc_b2p = jnp.pad(fc_b, (0, _PPAD - P)).reshape(1, -1)
    return _c2(out_pu, fc_Wp, fc_b2p)[:, :P]


# speedup vs baseline: 4.5325x; 1.3388x over previous
import functools

import jax
import jax.numpy as jnp
import numpy as np
from jax import lax
from jax.experimental import pallas as pl
from jax.experimental.pallas import tpu as pltpu
from jax.experimental.pallas import tpu_sc as plsc

P = 10000
U = 10000
H = 128
S = 100
B = 512
E = 320000

NC = 2
NS = 16
NW = NC * NS
LANES = 16

_MESH = plsc.VectorSubcoreMesh(core_axis_name="c", subcore_axis_name="s")


_EPT = E // NW
_C = 80
_NCH = _EPT // _C
_PT = 10240
_RPT = _PT // NS


@functools.partial(
    pl.kernel,
    out_type=jax.ShapeDtypeStruct((NC, _PT, H), jnp.float32),
    mesh=_MESH,
    scratch_types=[
        pltpu.VMEM((_C,), jnp.int32),
        pltpu.VMEM((_C,), jnp.int32),
        pltpu.VMEM((_C,), jnp.int32),
        pltpu.VMEM((_C,), jnp.int32),
        pltpu.VMEM((_EPT,), jnp.float32),
        pltpu.VMEM((_C, H), jnp.float32),
        pltpu.VMEM((_C, H), jnp.float32),
        pltpu.VMEM_SHARED((_PT, H), jnp.float32),
        pltpu.SemaphoreType.DMA,
        pltpu.SemaphoreType.DMA,
    ],
)
def _spmm(row_hbm, col_hbm, val_hbm, table_hbm, out_hbm,
          ridx0, ridx1, cidx0, cidx1, vals, rows0, rows1, acc, sem0, sem1):
    cid = lax.axis_index("c")
    sid = lax.axis_index("s")
    wid = sid * NC + cid
    ebase = wid * _EPT
    zv = jnp.zeros((LANES,), jnp.float32)

    def zrow(rr, c2):
        for cc in range(H // LANES):
            rows0[rr, pl.ds(cc * LANES, LANES)] = zv
        return c2

    lax.fori_loop(0, _C, zrow, 0)
    for k in range(_RPT // _C):
        pltpu.sync_copy(rows0, acc.at[pl.ds(sid * _RPT + k * _C, _C)])
    pltpu.sync_copy(val_hbm.at[pl.ds(ebase, _EPT)], vals)
    plsc.subcore_barrier()

    def scale(i, ridx, rows):

        @plsc.parallel_loop(0, _C // LANES, unroll=2)
        def grpscale(g):
            vv = vals[pl.ds(i * _C + g * LANES, LANES)]
            for j in range(LANES):
                bv = vv.at[jnp.full((LANES,), j, jnp.int32)].get(
                    mode="promise_in_bounds")
                rr = g * LANES + j
                for cc in range(H // LANES):
                    sl = (rr, pl.ds(cc * LANES, LANES))
                    rows[sl] = rows[sl] * bv

        pltpu.sync_copy(rows, acc.at[ridx], add=True)

    pltpu.sync_copy(col_hbm.at[pl.ds(ebase, _C)], cidx0)
    pltpu.async_copy(table_hbm.at[cidx0], rows0, sem0)

    def chunk2(i2, carry):
        i = i2 * 2
        pltpu.sync_copy(col_hbm.at[pl.ds(ebase + (i + 1) * _C, _C)], cidx1)
        pltpu.async_copy(table_hbm.at[cidx1], rows1, sem1)
        pltpu.sync_copy(row_hbm.at[pl.ds(ebase + i * _C, _C)], ridx0)
        pltpu.make_async_copy(table_hbm.at[cidx0], rows0, sem0).wait()
        scale(i, ridx0, rows0)

        pltpu.sync_copy(col_hbm.at[pl.ds(ebase + (i + 2) * _C, _C)], cidx0)
        pltpu.async_copy(table_hbm.at[cidx0], rows0, sem0)
        pltpu.sync_copy(row_hbm.at[pl.ds(ebase + (i + 1) * _C, _C)], ridx1)
        pltpu.make_async_copy(table_hbm.at[cidx1], rows1, sem1).wait()
        scale(i + 1, ridx1, rows1)
        return carry

    lax.fori_loop(0, (_NCH - 1) // 2, chunk2, 0)
    pltpu.sync_copy(row_hbm.at[pl.ds(ebase + (_NCH - 1) * _C, _C)], ridx0)
    pltpu.make_async_copy(table_hbm.at[cidx0], rows0, sem0).wait()
    scale(_NCH - 1, ridx0, rows0)

    plsc.subcore_barrier()
    pltpu.sync_copy(acc.at[pl.ds(sid * _RPT, _RPT)],
                    out_hbm.at[cid, pl.ds(sid * _RPT, _RPT)])


_ABLK = 2560


def _addtab_body(in_ref, out_ref):
    out_ref[...] = in_ref[0] + in_ref[1]


def _addtab(parts):
    return pl.pallas_call(
        _addtab_body,
        grid=(_PT // _ABLK,),
        in_specs=[pl.BlockSpec((2, _ABLK, H), lambda i: (0, i, 0))],
        out_specs=pl.BlockSpec((_ABLK, H), lambda i: (i, 0)),
        out_shape=jax.ShapeDtypeStruct((_PT, H), jnp.float32),
    )(parts)


def _make_gather1(n, c):
    n_w = n // NW
    nch = n_w // c

    @functools.partial(
        pl.kernel,
        out_type=jax.ShapeDtypeStruct((n, H), jnp.float32),
        mesh=_MESH,
        scratch_types=[
            pltpu.VMEM((c,), jnp.int32),
            pltpu.VMEM((c,), jnp.int32),
            pltpu.VMEM((c, H), jnp.float32),
            pltpu.VMEM((c, H), jnp.float32),
            pltpu.SemaphoreType.DMA,
            pltpu.SemaphoreType.DMA,
        ],
    )
    def g1(tab_hbm, idx_hbm, out_hbm, idx0, idx1, r0, r1, s0, s1):
        cid = lax.axis_index("c")
        sid = lax.axis_index("s")
        wid = sid * NC + cid
        base0 = wid * n_w
        pltpu.sync_copy(idx_hbm.at[pl.ds(base0, c)], idx0)
        pltpu.async_copy(tab_hbm.at[idx0], r0, s0)

        def chunk(i2, carry):
            i = i2 * 2
            pltpu.sync_copy(idx_hbm.at[pl.ds(base0 + (i + 1) * c, c)], idx1)
            pltpu.async_copy(tab_hbm.at[idx1], r1, s1)
            pltpu.make_async_copy(tab_hbm.at[idx0], r0, s0).wait()
            pltpu.sync_copy(r0, out_hbm.at[pl.ds(base0 + i * c, c)])

            @pl.when(i2 < nch // 2 - 1)
            def _pref():
                pltpu.sync_copy(idx_hbm.at[pl.ds(base0 + (i + 2) * c, c)],
                                idx0)
                pltpu.async_copy(tab_hbm.at[idx0], r0, s0)

            pltpu.make_async_copy(tab_hbm.at[idx1], r1, s1).wait()
            pltpu.sync_copy(r1, out_hbm.at[pl.ds(base0 + (i + 1) * c, c)])
            return carry

        lax.fori_loop(0, nch // 2, chunk, 0)

    return g1


_gather_x = _make_gather1(S * B, 80)


_BPT = B // NW


@functools.partial(
    pl.kernel,
    out_type=(jax.ShapeDtypeStruct((B, H), jnp.float32),
              jax.ShapeDtypeStruct((B, H), jnp.float32)),
    mesh=_MESH,
    scratch_types=[
        pltpu.VMEM((_BPT,), jnp.int32),
        pltpu.VMEM((_BPT, H), jnp.float32),
        pltpu.VMEM((_BPT, H), jnp.float32),
        pltpu.VMEM((_BPT, H), jnp.float32),
        pltpu.SemaphoreType.DMA,
        pltpu.SemaphoreType.DMA,
        pltpu.SemaphoreType.DMA,
    ],
)
def _gather_user(u0_hbm, u1_hbm, uemb_hbm, idx_hbm, up_hbm, pu_hbm,
                 idx_v, ra, rb, rc, s0, s1, s2):
    cid = lax.axis_index("c")
    sid = lax.axis_index("s")
    wid = sid * NC + cid
    base = wid * _BPT
    pltpu.sync_copy(idx_hbm.at[pl.ds(base, _BPT)], idx_v)
    cp0 = pltpu.async_copy(u0_hbm.at[idx_v], ra, s0)
    cp1 = pltpu.async_copy(u1_hbm.at[idx_v], rb, s1)
    cp2 = pltpu.async_copy(uemb_hbm.at[idx_v], rc, s2)
    cp0.wait()
    cp1.wait()
    cp2.wait()

    def rowadd(rr, c2):
        for cc in range(H // LANES):
            sl = (rr, pl.ds(cc * LANES, LANES))
            ra[sl] = ra[sl] + rb[sl]
        return c2

    lax.fori_loop(0, _BPT, rowadd, 0)
    pltpu.sync_copy(ra, up_hbm.at[pl.ds(base, _BPT)])
    pltpu.sync_copy(rc, pu_hbm.at[pl.ds(base, _BPT)])


_OMEGA = float(2.0 * np.pi / 86400.0)


def _c1_body(xe_ref, up_ref, pu_ref, t_ref, sx_ref, sy_ref, len_ref,
             h0_ref, wih_ref, whh_ref, bih_ref, bhh_ref, out_ref,
             h_s, acc_o, acc_w, tl, sxl, syl):
    i = pl.program_id(0)

    @pl.when(i == 0)
    def _init():
        h_s[...] = h0_ref[...]
        acc_o[...] = jnp.zeros_like(acc_o)
        acc_w[...] = jnp.zeros_like(acc_w)
        lm1 = len_ref[...] - 1
        ii = lax.broadcasted_iota(jnp.int32, (B, S), 1)
        selm = ii == lm1
        tl[...] = jnp.sum(jnp.where(selm, t_ref[...], 0.0), axis=1,
                          keepdims=True)
        sxl[...] = jnp.sum(jnp.where(selm, sx_ref[...], 0.0), axis=1,
                           keepdims=True)
        syl[...] = jnp.sum(jnp.where(selm, sy_ref[...], 0.0), axis=1,
                           keepdims=True)

    xe = xe_ref[0]
    d = up_ref[...] - xe
    sim = jnp.exp(-jnp.sqrt(jnp.sum(d * d, axis=1, keepdims=True) + 1e-12))

    gx = lax.dot_general(xe, wih_ref[...], (((1,), (1,)), ((), ())),
                         preferred_element_type=jnp.float32) + bih_ref[...]
    gh = lax.dot_general(h_s[...], whh_ref[...], (((1,), (1,)), ((), ())),
                         preferred_element_type=jnp.float32) + bhh_ref[...]
    xr, xz, xn = gx[:, :H], gx[:, H:2 * H], gx[:, 2 * H:]
    hr, hz, hn = gh[:, :H], gh[:, H:2 * H], gh[:, 2 * H:]
    r = jax.nn.sigmoid(xr + hr)
    z = jax.nn.sigmoid(xz + hz)
    nn = jnp.tanh(xn + r * hn)
    h_new = (1.0 - z) * nn + z * h_s[...]
    valid = i < len_ref[...]
    h_s[...] = jnp.where(valid, h_new, h_s[...])
    o = jnp.where(valid, h_new, 0.0)

    ii = lax.broadcasted_iota(jnp.int32, (B, S), 1)
    sel = ii == i
    t_i = jnp.sum(jnp.where(sel, t_ref[...], 0.0), axis=1, keepdims=True)
    sx_i = jnp.sum(jnp.where(sel, sx_ref[...], 0.0), axis=1, keepdims=True)
    sy_i = jnp.sum(jnp.where(sel, sy_ref[...], 0.0), axis=1, keepdims=True)

    dt = tl[...] - t_i
    a = (jnp.cos(dt * _OMEGA) + 1.0) * 0.5 * jnp.exp(dt * (-1e-5))
    dsx = sxl[...] - sx_i
    dsy = syl[...] - sy_i
    bw = jnp.exp(-jnp.sqrt(dsx * dsx + dsy * dsy + 1e-12))
    w = a * bw * sim
    w = jnp.where(valid, w, 0.0)
    acc_o[...] = acc_o[...] + w * o
    acc_w[...] = acc_w[...] + w

    @pl.when(i == S - 1)
    def _fin():
        out_ref[:, :H] = acc_o[...] / acc_w[...]
        out_ref[:, H:] = pu_ref[...]


def _c1(x_emb3, up, pu, t_bt, sx_bt, sy_bt, len_b1, h0,
        W_ih, W_hh, b_ih2, b_hh2):
    full = lambda shape: pl.BlockSpec(shape, lambda i: tuple(0 for _ in shape))
    return pl.pallas_call(
        _c1_body,
        grid=(S,),
        in_specs=[
            pl.BlockSpec((1, B, H), lambda i: (i, 0, 0)),
            full((B, H)), full((B, H)),
            full((B, S)), full((B, S)), full((B, S)),
            full((B, 1)), full((B, H)),
            full((3 * H, H)), full((3 * H, H)),
            full((1, 3 * H)), full((1, 3 * H)),
        ],
        out_specs=full((B, 2 * H)),
        out_shape=jax.ShapeDtypeStruct((B, 2 * H), jnp.float32),
        scratch_shapes=[
            pltpu.VMEM((B, H), jnp.float32),
            pltpu.VMEM((B, H), jnp.float32),
            pltpu.VMEM((B, 1), jnp.float32),
            pltpu.VMEM((B, 1), jnp.float32),
            pltpu.VMEM((B, 1), jnp.float32),
            pltpu.VMEM((B, 1), jnp.float32),
        ],
    )(x_emb3, up, pu, t_bt, sx_bt, sy_bt, len_b1, h0, W_ih, W_hh,
      b_ih2, b_hh2)


_PPAD = 10240
_CP = 2048


def _c2_body(pu_ref, w_ref, b_ref, out_ref):
    out_ref[...] = lax.dot_general(
        pu_ref[...], w_ref[...], (((1,), (1,)), ((), ())),
        preferred_element_type=jnp.float32) + b_ref[...]


def _c2(out_pu, fc_Wp, fc_b2p):
    return pl.pallas_call(
        _c2_body,
        grid=(_PPAD // _CP,),
        in_specs=[
            pl.BlockSpec((B, 2 * H), lambda i: (0, 0)),
            pl.BlockSpec((_CP, 2 * H), lambda i: (i, 0)),
            pl.BlockSpec((1, _CP), lambda i: (0, i)),
        ],
        out_specs=pl.BlockSpec((B, _CP), lambda i: (0, i)),
        out_shape=jax.ShapeDtypeStruct((B, _PPAD), jnp.float32),
    )(out_pu, fc_Wp, fc_b2p)


def kernel(x, t, s, y_t, y_s, h, active_user, length, emb, user_emb,
           W_ih, W_hh, b_ih, b_hh, fc_W, fc_b,
           trans_row, trans_col, trans_val, inter_row, inter_col, inter_val):
    x_flat = x.reshape(-1).astype(jnp.int32)
    au = active_user.reshape(-1).astype(jnp.int32)

    tp = _spmm(trans_row.astype(jnp.int32), trans_col.astype(jnp.int32),
               trans_val, emb)
    upar = _spmm(inter_row.astype(jnp.int32), inter_col.astype(jnp.int32),
                 inter_val, emb)

    x_emb = _gather_x(_addtab(tp), x_flat)
    up, pu = _gather_user(upar[0], upar[1], user_emb, au)

    out_pu = _c1(x_emb.reshape(S, B, H), up, pu,
                 t.T, s[..., 0].T, s[..., 1].T,
                 length.reshape(B, 1).astype(jnp.int32), h[0],
                 W_ih, W_hh, b_ih.reshape(1, -1), b_hh.reshape(1, -1))
    fc_Wp = jnp.pad(fc_W, ((0, _PPAD - P), (0, 0)))
    fc_b2p = jnp.pad(fc_b, (0, _PPAD - P)).reshape(1, -1)
    return _c2(out_pu, fc_Wp, fc_b2p)[:, :P]


# speedup vs baseline: 5.1063x; 1.1266x over previous
import functools

import jax
import jax.numpy as jnp
import numpy as np
from jax import lax
from jax.experimental import pallas as pl
from jax.experimental.pallas import tpu as pltpu
from jax.experimental.pallas import tpu_sc as plsc

P = 10000
U = 10000
H = 128
S = 100
B = 512
E = 320000

NC = 2
NS = 16
NW = NC * NS
LANES = 16

_MESH = plsc.VectorSubcoreMesh(core_axis_name="c", subcore_axis_name="s")


_EPT = E // NW
_C = 80
_NCH = _EPT // _C
_PT = 10240
_RPT = _PT // NS


@functools.partial(
    pl.kernel,
    out_type=jax.ShapeDtypeStruct((NC, _PT, H), jnp.float32),
    mesh=_MESH,
    scratch_types=[
        pltpu.VMEM((_C,), jnp.int32),
        pltpu.VMEM((_C,), jnp.int32),
        pltpu.VMEM((_C,), jnp.int32),
        pltpu.VMEM((_C,), jnp.int32),
        pltpu.VMEM((_EPT,), jnp.float32),
        pltpu.VMEM((_C, H), jnp.float32),
        pltpu.VMEM((_C, H), jnp.float32),
        pltpu.VMEM_SHARED((_PT, H), jnp.float32),
        pltpu.SemaphoreType.DMA,
        pltpu.SemaphoreType.DMA,
        pltpu.SemaphoreType.DMA,
        pltpu.SemaphoreType.DMA,
    ],
)
def _spmm(row_hbm, col_hbm, val_hbm, table_hbm, out_hbm,
          ridx0, ridx1, cidx0, cidx1, vals, rows0, rows1, acc,
          sem0, sem1, ssem0, ssem1):
    cid = lax.axis_index("c")
    sid = lax.axis_index("s")
    wid = sid * NC + cid
    ebase = wid * _EPT
    zv = jnp.zeros((LANES,), jnp.float32)

    def zrow(rr, c2):
        for cc in range(H // LANES):
            rows0[rr, pl.ds(cc * LANES, LANES)] = zv
        return c2

    lax.fori_loop(0, _C, zrow, 0)
    for k in range(_RPT // _C):
        pltpu.sync_copy(rows0, acc.at[pl.ds(sid * _RPT + k * _C, _C)])
    pltpu.sync_copy(val_hbm.at[pl.ds(ebase, _EPT)], vals)
    plsc.subcore_barrier()

    def scale(i, rows):

        @plsc.parallel_loop(0, _C // LANES, unroll=2)
        def grpscale(g):
            vv = vals[pl.ds(i * _C + g * LANES, LANES)]
            for j in range(LANES):
                bv = vv.at[jnp.full((LANES,), j, jnp.int32)].get(
                    mode="promise_in_bounds")
                rr = g * LANES + j
                for cc in range(H // LANES):
                    sl = (rr, pl.ds(cc * LANES, LANES))
                    rows[sl] = rows[sl] * bv

    pltpu.sync_copy(col_hbm.at[pl.ds(ebase, _C)], cidx0)
    pltpu.async_copy(table_hbm.at[cidx0], rows0, sem0)

    def chunk2(i2, carry):
        i = i2 * 2

        @pl.when(i2 > 0)
        def _drain1():
            pltpu.make_async_copy(rows1, acc.at[ridx1], ssem1).wait()

        pltpu.sync_copy(col_hbm.at[pl.ds(ebase + (i + 1) * _C, _C)], cidx1)
        pltpu.async_copy(table_hbm.at[cidx1], rows1, sem1)
        pltpu.sync_copy(row_hbm.at[pl.ds(ebase + i * _C, _C)], ridx0)
        pltpu.make_async_copy(table_hbm.at[cidx0], rows0, sem0).wait()
        scale(i, rows0)
        pltpu.async_copy(rows0, acc.at[ridx0], ssem0, add=True)

        pltpu.sync_copy(row_hbm.at[pl.ds(ebase + (i + 1) * _C, _C)], ridx1)
        pltpu.make_async_copy(table_hbm.at[cidx1], rows1, sem1).wait()
        scale(i + 1, rows1)
        pltpu.async_copy(rows1, acc.at[ridx1], ssem1, add=True)

        pltpu.make_async_copy(rows0, acc.at[ridx0], ssem0).wait()
        pltpu.sync_copy(col_hbm.at[pl.ds(ebase + (i + 2) * _C, _C)], cidx0)
        pltpu.async_copy(table_hbm.at[cidx0], rows0, sem0)
        return carry

    lax.fori_loop(0, (_NCH - 1) // 2, chunk2, 0)
    pltpu.make_async_copy(rows1, acc.at[ridx1], ssem1).wait()
    pltpu.sync_copy(row_hbm.at[pl.ds(ebase + (_NCH - 1) * _C, _C)], ridx0)
    pltpu.make_async_copy(table_hbm.at[cidx0], rows0, sem0).wait()
    scale(_NCH - 1, rows0)
    pltpu.sync_copy(rows0, acc.at[ridx0], add=True)

    plsc.subcore_barrier()
    pltpu.sync_copy(acc.at[pl.ds(sid * _RPT, _RPT)],
                    out_hbm.at[cid, pl.ds(sid * _RPT, _RPT)])


_ABLK = 2560


def _addtab_body(in_ref, out_ref):
    out_ref[...] = in_ref[0] + in_ref[1]


def _addtab(parts):
    return pl.pallas_call(
        _addtab_body,
        grid=(_PT // _ABLK,),
        in_specs=[pl.BlockSpec((2, _ABLK, H), lambda i: (0, i, 0))],
        out_specs=pl.BlockSpec((_ABLK, H), lambda i: (i, 0)),
        out_shape=jax.ShapeDtypeStruct((_PT, H), jnp.float32),
    )(parts)


def _make_gather1(n, c):
    n_w = n // NW
    nch = n_w // c

    @functools.partial(
        pl.kernel,
        out_type=jax.ShapeDtypeStruct((n, H), jnp.float32),
        mesh=_MESH,
        scratch_types=[
            pltpu.VMEM((c,), jnp.int32),
            pltpu.VMEM((c,), jnp.int32),
            pltpu.VMEM((c, H), jnp.float32),
            pltpu.VMEM((c, H), jnp.float32),
            pltpu.SemaphoreType.DMA,
            pltpu.SemaphoreType.DMA,
        ],
    )
    def g1(tab_hbm, idx_hbm, out_hbm, idx0, idx1, r0, r1, s0, s1):
        cid = lax.axis_index("c")
        sid = lax.axis_index("s")
        wid = sid * NC + cid
        base0 = wid * n_w
        pltpu.sync_copy(idx_hbm.at[pl.ds(base0, c)], idx0)
        pltpu.async_copy(tab_hbm.at[idx0], r0, s0)

        def chunk(i2, carry):
            i = i2 * 2
            pltpu.sync_copy(idx_hbm.at[pl.ds(base0 + (i + 1) * c, c)], idx1)
            pltpu.async_copy(tab_hbm.at[idx1], r1, s1)
            pltpu.make_async_copy(tab_hbm.at[idx0], r0, s0).wait()
            pltpu.sync_copy(r0, out_hbm.at[pl.ds(base0 + i * c, c)])

            @pl.when(i2 < nch // 2 - 1)
            def _pref():
                pltpu.sync_copy(idx_hbm.at[pl.ds(base0 + (i + 2) * c, c)],
                                idx0)
                pltpu.async_copy(tab_hbm.at[idx0], r0, s0)

            pltpu.make_async_copy(tab_hbm.at[idx1], r1, s1).wait()
            pltpu.sync_copy(r1, out_hbm.at[pl.ds(base0 + (i + 1) * c, c)])
            return carry

        lax.fori_loop(0, nch // 2, chunk, 0)

    return g1


_gather_x = _make_gather1(S * B, 80)


_BPT = B // NW


@functools.partial(
    pl.kernel,
    out_type=(jax.ShapeDtypeStruct((B, H), jnp.float32),
              jax.ShapeDtypeStruct((B, H), jnp.float32)),
    mesh=_MESH,
    scratch_types=[
        pltpu.VMEM((_BPT,), jnp.int32),
        pltpu.VMEM((_BPT, H), jnp.float32),
        pltpu.VMEM((_BPT, H), jnp.float32),
        pltpu.VMEM((_BPT, H), jnp.float32),
        pltpu.SemaphoreType.DMA,
        pltpu.SemaphoreType.DMA,
        pltpu.SemaphoreType.DMA,
    ],
)
def _gather_user(u0_hbm, u1_hbm, uemb_hbm, idx_hbm, up_hbm, pu_hbm,
                 idx_v, ra, rb, rc, s0, s1, s2):
    cid = lax.axis_index("c")
    sid = lax.axis_index("s")
    wid = sid * NC + cid
    base = wid * _BPT
    pltpu.sync_copy(idx_hbm.at[pl.ds(base, _BPT)], idx_v)
    cp0 = pltpu.async_copy(u0_hbm.at[idx_v], ra, s0)
    cp1 = pltpu.async_copy(u1_hbm.at[idx_v], rb, s1)
    cp2 = pltpu.async_copy(uemb_hbm.at[idx_v], rc, s2)
    cp0.wait()
    cp1.wait()
    cp2.wait()

    def rowadd(rr, c2):
        for cc in range(H // LANES):
            sl = (rr, pl.ds(cc * LANES, LANES))
            ra[sl] = ra[sl] + rb[sl]
        return c2

    lax.fori_loop(0, _BPT, rowadd, 0)
    pltpu.sync_copy(ra, up_hbm.at[pl.ds(base, _BPT)])
    pltpu.sync_copy(rc, pu_hbm.at[pl.ds(base, _BPT)])


_OMEGA = float(2.0 * np.pi / 86400.0)


def _c1_body(xe_ref, up_ref, pu_ref, t_ref, sx_ref, sy_ref, len_ref,
             h0_ref, wih_ref, whh_ref, bih_ref, bhh_ref, out_ref,
             h_s, acc_o, acc_w, tl, sxl, syl):
    i = pl.program_id(0)

    @pl.when(i == 0)
    def _init():
        h_s[...] = h0_ref[...]
        acc_o[...] = jnp.zeros_like(acc_o)
        acc_w[...] = jnp.zeros_like(acc_w)
        lm1 = len_ref[...] - 1
        ii = lax.broadcasted_iota(jnp.int32, (B, S), 1)
        selm = ii == lm1
        tl[...] = jnp.sum(jnp.where(selm, t_ref[...], 0.0), axis=1,
                          keepdims=True)
        sxl[...] = jnp.sum(jnp.where(selm, sx_ref[...], 0.0), axis=1,
                           keepdims=True)
        syl[...] = jnp.sum(jnp.where(selm, sy_ref[...], 0.0), axis=1,
                           keepdims=True)

    xe = xe_ref[0]
    d = up_ref[...] - xe
    sim = jnp.exp(-jnp.sqrt(jnp.sum(d * d, axis=1, keepdims=True) + 1e-12))

    gx = lax.dot_general(xe.astype(jnp.bfloat16), wih_ref[...],
                         (((1,), (1,)), ((), ())),
                         preferred_element_type=jnp.float32) + bih_ref[...]
    gh = lax.dot_general(h_s[...].astype(jnp.bfloat16), whh_ref[...],
                         (((1,), (1,)), ((), ())),
                         preferred_element_type=jnp.float32) + bhh_ref[...]
    xr, xz, xn = gx[:, :H], gx[:, H:2 * H], gx[:, 2 * H:]
    hr, hz, hn = gh[:, :H], gh[:, H:2 * H], gh[:, 2 * H:]
    r = jax.nn.sigmoid(xr + hr)
    z = jax.nn.sigmoid(xz + hz)
    nn = jnp.tanh(xn + r * hn)
    h_new = (1.0 - z) * nn + z * h_s[...]
    valid = i < len_ref[...]
    h_s[...] = jnp.where(valid, h_new, h_s[...])
    o = jnp.where(valid, h_new, 0.0)

    ii = lax.broadcasted_iota(jnp.int32, (B, S), 1)
    sel = ii == i
    t_i = jnp.sum(jnp.where(sel, t_ref[...], 0.0), axis=1, keepdims=True)
    sx_i = jnp.sum(jnp.where(sel, sx_ref[...], 0.0), axis=1, keepdims=True)
    sy_i = jnp.sum(jnp.where(sel, sy_ref[...], 0.0), axis=1, keepdims=True)

    dt = tl[...] - t_i
    a = (jnp.cos(dt * _OMEGA) + 1.0) * 0.5 * jnp.exp(dt * (-1e-5))
    dsx = sxl[...] - sx_i
    dsy = syl[...] - sy_i
    bw = jnp.exp(-jnp.sqrt(dsx * dsx + dsy * dsy + 1e-12))
    w = a * bw * sim
    w = jnp.where(valid, w, 0.0)
    acc_o[...] = acc_o[...] + w * o
    acc_w[...] = acc_w[...] + w

    @pl.when(i == S - 1)
    def _fin():
        out_ref[:, :H] = acc_o[...] / acc_w[...]
        out_ref[:, H:] = pu_ref[...]


def _c1(x_emb3, up, pu, t_bt, sx_bt, sy_bt, len_b1, h0,
        W_ih, W_hh, b_ih2, b_hh2):
    full = lambda shape: pl.BlockSpec(shape, lambda i: tuple(0 for _ in shape))
    return pl.pallas_call(
        _c1_body,
        grid=(S,),
        in_specs=[
            pl.BlockSpec((1, B, H), lambda i: (i, 0, 0)),
            full((B, H)), full((B, H)),
            full((B, S)), full((B, S)), full((B, S)),
            full((B, 1)), full((B, H)),
            full((3 * H, H)), full((3 * H, H)),
            full((1, 3 * H)), full((1, 3 * H)),
        ],
        out_specs=full((B, 2 * H)),
        out_shape=jax.ShapeDtypeStruct((B, 2 * H), jnp.float32),
        scratch_shapes=[
            pltpu.VMEM((B, H), jnp.float32),
            pltpu.VMEM((B, H), jnp.float32),
            pltpu.VMEM((B, 1), jnp.float32),
            pltpu.VMEM((B, 1), jnp.float32),
            pltpu.VMEM((B, 1), jnp.float32),
            pltpu.VMEM((B, 1), jnp.float32),
        ],
    )(x_emb3, up, pu, t_bt, sx_bt, sy_bt, len_b1, h0, W_ih, W_hh,
      b_ih2, b_hh2)


_PPAD = 10240
_CP = 2048


def _c2_body(pu_ref, w_ref, b_ref, out_ref):
    out_ref[...] = lax.dot_general(
        pu_ref[...].astype(jnp.bfloat16), w_ref[...],
        (((1,), (1,)), ((), ())),
        preferred_element_type=jnp.float32) + b_ref[...]


def _c2(out_pu, fc_Wp, fc_b2p):
    return pl.pallas_call(
        _c2_body,
        grid=(_PPAD // _CP,),
        in_specs=[
            pl.BlockSpec((B, 2 * H), lambda i: (0, 0)),
            pl.BlockSpec((_CP, 2 * H), lambda i: (i, 0)),
            pl.BlockSpec((1, _CP), lambda i: (0, i)),
        ],
        out_specs=pl.BlockSpec((B, _CP), lambda i: (0, i)),
        out_shape=jax.ShapeDtypeStruct((B, _PPAD), jnp.float32),
    )(out_pu, fc_Wp, fc_b2p)


def kernel(x, t, s, y_t, y_s, h, active_user, length, emb, user_emb,
           W_ih, W_hh, b_ih, b_hh, fc_W, fc_b,
           trans_row, trans_col, trans_val, inter_row, inter_col, inter_val):
    x_flat = x.reshape(-1).astype(jnp.int32)
    au = active_user.reshape(-1).astype(jnp.int32)

    tp = _spmm(trans_row.astype(jnp.int32), trans_col.astype(jnp.int32),
               trans_val, emb)
    upar = _spmm(inter_row.astype(jnp.int32), inter_col.astype(jnp.int32),
                 inter_val, emb)

    x_emb = _gather_x(_addtab(tp), x_flat)
    up, pu = _gather_user(upar[0], upar[1], user_emb, au)

    out_pu = _c1(x_emb.reshape(S, B, H), up, pu,
                 t.T, s[..., 0].T, s[..., 1].T,
                 length.reshape(B, 1).astype(jnp.int32), h[0],
                 W_ih.astype(jnp.bfloat16), W_hh.astype(jnp.bfloat16),
                 b_ih.reshape(1, -1), b_hh.reshape(1, -1))
    fc_Wp = jnp.pad(fc_W, ((0, _PPAD - P), (0, 0))).astype(jnp.bfloat16)
    fc_b2p = jnp.pad(fc_b, (0, _PPAD - P)).reshape(1, -1)
    return _c2(out_pu, fc_Wp, fc_b2p)[:, :P]


# speedup vs baseline: 5.2362x; 1.0254x over previous
import functools

import jax
import jax.numpy as jnp
import numpy as np
from jax import lax
from jax.experimental import pallas as pl
from jax.experimental.pallas import tpu as pltpu
from jax.experimental.pallas import tpu_sc as plsc

P = 10000
U = 10000
H = 128
S = 100
B = 512
E = 320000

NC = 2
NS = 16
NW = NC * NS
LANES = 16

_MESH = plsc.VectorSubcoreMesh(core_axis_name="c", subcore_axis_name="s")


_EPT = E // NW
_C = 80
_NCH = _EPT // _C
_PT = 10240
_RPT = _PT // NS


@functools.partial(
    pl.kernel,
    out_type=jax.ShapeDtypeStruct((NC, _PT, H), jnp.float32),
    mesh=_MESH,
    scratch_types=[
        pltpu.VMEM((_C,), jnp.int32),
        pltpu.VMEM((_C,), jnp.int32),
        pltpu.VMEM((_C,), jnp.int32),
        pltpu.VMEM((_C,), jnp.int32),
        pltpu.VMEM((_EPT,), jnp.float32),
        pltpu.VMEM((_C, H), jnp.float32),
        pltpu.VMEM((_C, H), jnp.float32),
        pltpu.VMEM_SHARED((_PT, H), jnp.float32),
        pltpu.SemaphoreType.DMA,
        pltpu.SemaphoreType.DMA,
        pltpu.SemaphoreType.DMA,
        pltpu.SemaphoreType.DMA,
    ],
)
def _spmm(row_hbm, col_hbm, val_hbm, table_hbm, out_hbm,
          ridx0, ridx1, cidx0, cidx1, vals, rows0, rows1, acc,
          sem0, sem1, ssem0, ssem1):
    cid = lax.axis_index("c")
    sid = lax.axis_index("s")
    wid = sid * NC + cid
    ebase = wid * _EPT
    zv = jnp.zeros((LANES,), jnp.float32)

    def zrow(rr, c2):
        for cc in range(H // LANES):
            rows0[rr, pl.ds(cc * LANES, LANES)] = zv
        return c2

    lax.fori_loop(0, _C, zrow, 0)
    for k in range(_RPT // _C):
        pltpu.sync_copy(rows0, acc.at[pl.ds(sid * _RPT + k * _C, _C)])
    pltpu.sync_copy(val_hbm.at[pl.ds(ebase, _EPT)], vals)
    plsc.subcore_barrier()

    def scale(i, rows):

        @plsc.parallel_loop(0, _C // LANES, unroll=2)
        def grpscale(g):
            vv = vals[pl.ds(i * _C + g * LANES, LANES)]
            for j in range(LANES):
                bv = vv.at[jnp.full((LANES,), j, jnp.int32)].get(
                    mode="promise_in_bounds")
                rr = g * LANES + j
                for cc in range(H // LANES):
                    sl = (rr, pl.ds(cc * LANES, LANES))
                    rows[sl] = rows[sl] * bv

    pltpu.sync_copy(col_hbm.at[pl.ds(ebase, _C)], cidx0)
    pltpu.async_copy(table_hbm.at[cidx0], rows0, sem0)

    def chunk2(i2, carry):
        i = i2 * 2

        @pl.when(i2 > 0)
        def _drain1():
            pltpu.make_async_copy(rows1, acc.at[ridx1], ssem1).wait()

        pltpu.sync_copy(col_hbm.at[pl.ds(ebase + (i + 1) * _C, _C)], cidx1)
        pltpu.async_copy(table_hbm.at[cidx1], rows1, sem1)
        pltpu.sync_copy(row_hbm.at[pl.ds(ebase + i * _C, _C)], ridx0)
        pltpu.make_async_copy(table_hbm.at[cidx0], rows0, sem0).wait()
        scale(i, rows0)
        pltpu.async_copy(rows0, acc.at[ridx0], ssem0, add=True)

        pltpu.sync_copy(row_hbm.at[pl.ds(ebase + (i + 1) * _C, _C)], ridx1)
        pltpu.make_async_copy(table_hbm.at[cidx1], rows1, sem1).wait()
        scale(i + 1, rows1)
        pltpu.async_copy(rows1, acc.at[ridx1], ssem1, add=True)

        pltpu.make_async_copy(rows0, acc.at[ridx0], ssem0).wait()
        pltpu.sync_copy(col_hbm.at[pl.ds(ebase + (i + 2) * _C, _C)], cidx0)
        pltpu.async_copy(table_hbm.at[cidx0], rows0, sem0)
        return carry

    lax.fori_loop(0, (_NCH - 1) // 2, chunk2, 0)
    pltpu.make_async_copy(rows1, acc.at[ridx1], ssem1).wait()
    pltpu.sync_copy(row_hbm.at[pl.ds(ebase + (_NCH - 1) * _C, _C)], ridx0)
    pltpu.make_async_copy(table_hbm.at[cidx0], rows0, sem0).wait()
    scale(_NCH - 1, rows0)
    pltpu.sync_copy(rows0, acc.at[ridx0], add=True)

    plsc.subcore_barrier()
    pltpu.sync_copy(acc.at[pl.ds(sid * _RPT, _RPT)],
                    out_hbm.at[cid, pl.ds(sid * _RPT, _RPT)])


_ABLK = 2560


def _addtab_body(in_ref, out_ref):
    out_ref[...] = in_ref[0] + in_ref[1]


def _addtab(parts):
    return pl.pallas_call(
        _addtab_body,
        grid=(_PT // _ABLK,),
        in_specs=[pl.BlockSpec((2, _ABLK, H), lambda i: (0, i, 0))],
        out_specs=pl.BlockSpec((_ABLK, H), lambda i: (i, 0)),
        out_shape=jax.ShapeDtypeStruct((_PT, H), jnp.float32),
    )(parts)


def _make_gather1(n, c):
    n_w = n // NW
    nch = n_w // c

    @functools.partial(
        pl.kernel,
        out_type=jax.ShapeDtypeStruct((n, H), jnp.float32),
        mesh=_MESH,
        scratch_types=[
            pltpu.VMEM((c,), jnp.int32),
            pltpu.VMEM((c,), jnp.int32),
            pltpu.VMEM((c, H), jnp.float32),
            pltpu.VMEM((c, H), jnp.float32),
            pltpu.SemaphoreType.DMA,
            pltpu.SemaphoreType.DMA,
        ],
    )
    def g1(tab_hbm, idx_hbm, out_hbm, idx0, idx1, r0, r1, s0, s1):
        cid = lax.axis_index("c")
        sid = lax.axis_index("s")
        wid = sid * NC + cid
        base0 = wid * n_w
        pltpu.sync_copy(idx_hbm.at[pl.ds(base0, c)], idx0)
        pltpu.async_copy(tab_hbm.at[idx0], r0, s0)

        def chunk(i2, carry):
            i = i2 * 2
            pltpu.sync_copy(idx_hbm.at[pl.ds(base0 + (i + 1) * c, c)], idx1)
            pltpu.async_copy(tab_hbm.at[idx1], r1, s1)
            pltpu.make_async_copy(tab_hbm.at[idx0], r0, s0).wait()
            pltpu.sync_copy(r0, out_hbm.at[pl.ds(base0 + i * c, c)])

            @pl.when(i2 < nch // 2 - 1)
            def _pref():
                pltpu.sync_copy(idx_hbm.at[pl.ds(base0 + (i + 2) * c, c)],
                                idx0)
                pltpu.async_copy(tab_hbm.at[idx0], r0, s0)

            pltpu.make_async_copy(tab_hbm.at[idx1], r1, s1).wait()
            pltpu.sync_copy(r1, out_hbm.at[pl.ds(base0 + (i + 1) * c, c)])
            return carry

        lax.fori_loop(0, nch // 2, chunk, 0)

    return g1


_gather_x = _make_gather1(S * B, 80)


_BPT = B // NW


@functools.partial(
    pl.kernel,
    out_type=(jax.ShapeDtypeStruct((B, H), jnp.float32),
              jax.ShapeDtypeStruct((B, H), jnp.float32)),
    mesh=_MESH,
    scratch_types=[
        pltpu.VMEM((_BPT,), jnp.int32),
        pltpu.VMEM((_BPT, H), jnp.float32),
        pltpu.VMEM((_BPT, H), jnp.float32),
        pltpu.VMEM((_BPT, H), jnp.float32),
        pltpu.SemaphoreType.DMA,
        pltpu.SemaphoreType.DMA,
        pltpu.SemaphoreType.DMA,
    ],
)
def _gather_user(u0_hbm, u1_hbm, uemb_hbm, idx_hbm, up_hbm, pu_hbm,
                 idx_v, ra, rb, rc, s0, s1, s2):
    cid = lax.axis_index("c")
    sid = lax.axis_index("s")
    wid = sid * NC + cid
    base = wid * _BPT
    pltpu.sync_copy(idx_hbm.at[pl.ds(base, _BPT)], idx_v)
    cp0 = pltpu.async_copy(u0_hbm.at[idx_v], ra, s0)
    cp1 = pltpu.async_copy(u1_hbm.at[idx_v], rb, s1)
    cp2 = pltpu.async_copy(uemb_hbm.at[idx_v], rc, s2)
    cp0.wait()
    cp1.wait()
    cp2.wait()

    def rowadd(rr, c2):
        for cc in range(H // LANES):
            sl = (rr, pl.ds(cc * LANES, LANES))
            ra[sl] = ra[sl] + rb[sl]
        return c2

    lax.fori_loop(0, _BPT, rowadd, 0)
    pltpu.sync_copy(ra, up_hbm.at[pl.ds(base, _BPT)])
    pltpu.sync_copy(rc, pu_hbm.at[pl.ds(base, _BPT)])


_OMEGA = float(2.0 * np.pi / 86400.0)
_KS = 5


def _c1_body(xe_ref, up_ref, pu_ref, t_ref, sx_ref, sy_ref, len_ref,
             h0_ref, wih_ref, whh_ref, bih_ref, bhh_ref, out_ref,
             h_s, acc_o, acc_w, tl, sxl, syl):
    pid = pl.program_id(0)

    @pl.when(pid == 0)
    def _init():
        h_s[...] = h0_ref[...]
        acc_o[...] = jnp.zeros_like(acc_o)
        acc_w[...] = jnp.zeros_like(acc_w)
        lm1 = len_ref[...] - 1
        ii = lax.broadcasted_iota(jnp.int32, (B, S), 1)
        selm = ii == lm1
        tl[...] = jnp.sum(jnp.where(selm, t_ref[...], 0.0), axis=1,
                          keepdims=True)
        sxl[...] = jnp.sum(jnp.where(selm, sx_ref[...], 0.0), axis=1,
                           keepdims=True)
        syl[...] = jnp.sum(jnp.where(selm, sy_ref[...], 0.0), axis=1,
                           keepdims=True)

    ii = lax.broadcasted_iota(jnp.int32, (B, S), 1)
    for k in range(_KS):
        i = pid * _KS + k
        xe = xe_ref[k]
        d = up_ref[...] - xe
        sim = jnp.exp(-jnp.sqrt(jnp.sum(d * d, axis=1, keepdims=True)
                                + 1e-12))

        gx = lax.dot_general(xe.astype(jnp.bfloat16), wih_ref[...],
                             (((1,), (1,)), ((), ())),
                             preferred_element_type=jnp.float32) + bih_ref[...]
        gh = lax.dot_general(h_s[...].astype(jnp.bfloat16), whh_ref[...],
                             (((1,), (1,)), ((), ())),
                             preferred_element_type=jnp.float32) + bhh_ref[...]
        xr, xz, xn = gx[:, :H], gx[:, H:2 * H], gx[:, 2 * H:]
        hr, hz, hn = gh[:, :H], gh[:, H:2 * H], gh[:, 2 * H:]
        r = jax.nn.sigmoid(xr + hr)
        z = jax.nn.sigmoid(xz + hz)
        nn = jnp.tanh(xn + r * hn)
        h_new = (1.0 - z) * nn + z * h_s[...]
        valid = i < len_ref[...]
        h_s[...] = jnp.where(valid, h_new, h_s[...])
        o = jnp.where(valid, h_new, 0.0)

        sel = ii == i
        t_i = jnp.sum(jnp.where(sel, t_ref[...], 0.0), axis=1, keepdims=True)
        sx_i = jnp.sum(jnp.where(sel, sx_ref[...], 0.0), axis=1,
                       keepdims=True)
        sy_i = jnp.sum(jnp.where(sel, sy_ref[...], 0.0), axis=1,
                       keepdims=True)

        dt = tl[...] - t_i
        a = (jnp.cos(dt * _OMEGA) + 1.0) * 0.5 * jnp.exp(dt * (-1e-5))
        dsx = sxl[...] - sx_i
        dsy = syl[...] - sy_i
        bw = jnp.exp(-jnp.sqrt(dsx * dsx + dsy * dsy + 1e-12))
        w = a * bw * sim
        w = jnp.where(valid, w, 0.0)
        acc_o[...] = acc_o[...] + w * o
        acc_w[...] = acc_w[...] + w

    @pl.when(pid == S // _KS - 1)
    def _fin():
        out_ref[:, :H] = acc_o[...] / acc_w[...]
        out_ref[:, H:] = pu_ref[...]


def _c1(x_emb3, up, pu, t_bt, sx_bt, sy_bt, len_b1, h0,
        W_ih, W_hh, b_ih2, b_hh2):
    full = lambda shape: pl.BlockSpec(shape, lambda i: tuple(0 for _ in shape))
    return pl.pallas_call(
        _c1_body,
        grid=(S // _KS,),
        in_specs=[
            pl.BlockSpec((_KS, B, H), lambda i: (i, 0, 0)),
            full((B, H)), full((B, H)),
            full((B, S)), full((B, S)), full((B, S)),
            full((B, 1)), full((B, H)),
            full((3 * H, H)), full((3 * H, H)),
            full((1, 3 * H)), full((1, 3 * H)),
        ],
        out_specs=full((B, 2 * H)),
        out_shape=jax.ShapeDtypeStruct((B, 2 * H), jnp.float32),
        scratch_shapes=[
            pltpu.VMEM((B, H), jnp.float32),
            pltpu.VMEM((B, H), jnp.float32),
            pltpu.VMEM((B, 1), jnp.float32),
            pltpu.VMEM((B, 1), jnp.float32),
            pltpu.VMEM((B, 1), jnp.float32),
            pltpu.VMEM((B, 1), jnp.float32),
        ],
    )(x_emb3, up, pu, t_bt, sx_bt, sy_bt, len_b1, h0, W_ih, W_hh,
      b_ih2, b_hh2)


_PPAD = 10240
_CP = 2048


def _c2_body(pu_ref, w_ref, b_ref, out_ref):
    out_ref[...] = lax.dot_general(
        pu_ref[...].astype(jnp.bfloat16), w_ref[...],
        (((1,), (1,)), ((), ())),
        preferred_element_type=jnp.float32) + b_ref[...]


def _c2(out_pu, fc_Wp, fc_b2p):
    return pl.pallas_call(
        _c2_body,
        grid=(_PPAD // _CP,),
        in_specs=[
            pl.BlockSpec((B, 2 * H), lambda i: (0, 0)),
            pl.BlockSpec((_CP, 2 * H), lambda i: (i, 0)),
            pl.BlockSpec((1, _CP), lambda i: (0, i)),
        ],
        out_specs=pl.BlockSpec((B, _CP), lambda i: (0, i)),
        out_shape=jax.ShapeDtypeStruct((B, _PPAD), jnp.float32),
    )(out_pu, fc_Wp, fc_b2p)


def kernel(x, t, s, y_t, y_s, h, active_user, length, emb, user_emb,
           W_ih, W_hh, b_ih, b_hh, fc_W, fc_b,
           trans_row, trans_col, trans_val, inter_row, inter_col, inter_val):
    x_flat = x.reshape(-1).astype(jnp.int32)
    au = active_user.reshape(-1).astype(jnp.int32)

    tp = _spmm(trans_row.astype(jnp.int32), trans_col.astype(jnp.int32),
               trans_val, emb)
    upar = _spmm(inter_row.astype(jnp.int32), inter_col.astype(jnp.int32),
                 inter_val, emb)

    x_emb = _gather_x(_addtab(tp), x_flat)
    up, pu = _gather_user(upar[0], upar[1], user_emb, au)

    out_pu = _c1(x_emb.reshape(S, B, H), up, pu,
                 t.T, s[..., 0].T, s[..., 1].T,
                 length.reshape(B, 1).astype(jnp.int32), h[0],
                 W_ih.astype(jnp.bfloat16), W_hh.astype(jnp.bfloat16),
                 b_ih.reshape(1, -1), b_hh.reshape(1, -1))
    fc_Wp = jnp.pad(fc_W, ((0, _PPAD - P), (0, 0))).astype(jnp.bfloat16)
    fc_b2p = jnp.pad(fc_b, (0, _PPAD - P)).reshape(1, -1)
    return _c2(out_pu, fc_Wp, fc_b2p)[:, :P]


# speedup vs baseline: 5.8186x; 1.1112x over previous
import functools

import jax
import jax.numpy as jnp
import numpy as np
from jax import lax
from jax.experimental import pallas as pl
from jax.experimental.pallas import tpu as pltpu
from jax.experimental.pallas import tpu_sc as plsc

P = 10000
U = 10000
H = 128
S = 100
B = 512
E = 320000

NC = 2
NS = 16
NW = NC * NS
LANES = 16

_MESH = plsc.VectorSubcoreMesh(core_axis_name="c", subcore_axis_name="s")


_EPT = E // NW
_C = 80
_NCH = _EPT // _C
_PT = 10240
_RPT = _PT // NS


@functools.partial(
    pl.kernel,
    out_type=jax.ShapeDtypeStruct((NC, _PT, H), jnp.float32),
    mesh=_MESH,
    scratch_types=[
        pltpu.VMEM((_C,), jnp.int32),
        pltpu.VMEM((_C,), jnp.int32),
        pltpu.VMEM((_EPT,), jnp.int32),
        pltpu.VMEM((_EPT,), jnp.float32),
        pltpu.VMEM((_C, H), jnp.float32),
        pltpu.VMEM((_C, H), jnp.float32),
        pltpu.VMEM_SHARED((_PT, H), jnp.float32),
        pltpu.SemaphoreType.DMA,
        pltpu.SemaphoreType.DMA,
        pltpu.SemaphoreType.DMA,
        pltpu.SemaphoreType.DMA,
    ],
)
def _spmm(row_hbm, col_hbm, val_hbm, table_hbm, out_hbm,
          ridx0, ridx1, cols, vals, rows0, rows1, acc,
          sem0, sem1, ssem0, ssem1):
    cid = lax.axis_index("c")
    sid = lax.axis_index("s")
    wid = sid * NC + cid
    ebase = wid * _EPT
    zv = jnp.zeros((LANES,), jnp.float32)

    def zrow(rr, c2):
        for cc in range(H // LANES):
            rows0[rr, pl.ds(cc * LANES, LANES)] = zv
        return c2

    lax.fori_loop(0, _C, zrow, 0)
    for k in range(_RPT // _C):
        pltpu.sync_copy(rows0, acc.at[pl.ds(sid * _RPT + k * _C, _C)])
    pltpu.sync_copy(val_hbm.at[pl.ds(ebase, _EPT)], vals)
    pltpu.sync_copy(col_hbm.at[pl.ds(ebase, _EPT)], cols)
    plsc.subcore_barrier()

    def scale(i, rows):

        @plsc.parallel_loop(0, _C // LANES, unroll=2)
        def grpscale(g):
            vv = vals[pl.ds(i * _C + g * LANES, LANES)]
            for j in range(LANES):
                bv = vv.at[jnp.full((LANES,), j, jnp.int32)].get(
                    mode="promise_in_bounds")
                rr = g * LANES + j
                for cc in range(H // LANES):
                    sl = (rr, pl.ds(cc * LANES, LANES))
                    rows[sl] = rows[sl] * bv

    pltpu.async_copy(table_hbm.at[cols.at[pl.ds(0, _C)]], rows0, sem0)

    def chunk2(i2, carry):
        i = i2 * 2

        @pl.when(i2 > 0)
        def _drain1():
            pltpu.make_async_copy(rows1, acc.at[ridx1], ssem1).wait()

        pltpu.async_copy(table_hbm.at[cols.at[pl.ds((i + 1) * _C, _C)]],
                         rows1, sem1)
        pltpu.sync_copy(row_hbm.at[pl.ds(ebase + i * _C, _C)], ridx0)
        pltpu.make_async_copy(table_hbm.at[cols.at[pl.ds(0, _C)]],
                              rows0, sem0).wait()
        scale(i, rows0)
        pltpu.async_copy(rows0, acc.at[ridx0], ssem0, add=True)

        pltpu.sync_copy(row_hbm.at[pl.ds(ebase + (i + 1) * _C, _C)], ridx1)
        pltpu.make_async_copy(table_hbm.at[cols.at[pl.ds(0, _C)]],
                              rows1, sem1).wait()
        scale(i + 1, rows1)
        pltpu.async_copy(rows1, acc.at[ridx1], ssem1, add=True)

        pltpu.make_async_copy(rows0, acc.at[ridx0], ssem0).wait()
        pltpu.async_copy(table_hbm.at[cols.at[pl.ds((i + 2) * _C, _C)]],
                         rows0, sem0)
        return carry

    lax.fori_loop(0, (_NCH - 1) // 2, chunk2, 0)
    pltpu.make_async_copy(rows1, acc.at[ridx1], ssem1).wait()
    pltpu.sync_copy(row_hbm.at[pl.ds(ebase + (_NCH - 1) * _C, _C)], ridx0)
    pltpu.make_async_copy(table_hbm.at[cols.at[pl.ds(0, _C)]],
                          rows0, sem0).wait()
    scale(_NCH - 1, rows0)
    pltpu.sync_copy(rows0, acc.at[ridx0], add=True)

    plsc.subcore_barrier()
    pltpu.sync_copy(acc.at[pl.ds(sid * _RPT, _RPT)],
                    out_hbm.at[cid, pl.ds(sid * _RPT, _RPT)])


_ABLK = 2560


def _addtab_body(in_ref, out_ref):
    out_ref[...] = in_ref[0] + in_ref[1]


def _addtab(parts):
    return pl.pallas_call(
        _addtab_body,
        grid=(_PT // _ABLK,),
        in_specs=[pl.BlockSpec((2, _ABLK, H), lambda i: (0, i, 0))],
        out_specs=pl.BlockSpec((_ABLK, H), lambda i: (i, 0)),
        out_shape=jax.ShapeDtypeStruct((_PT, H), jnp.float32),
    )(parts)


def _make_gather1(n, c):
    n_w = n // NW
    nch = n_w // c

    @functools.partial(
        pl.kernel,
        out_type=jax.ShapeDtypeStruct((n, H), jnp.float32),
        mesh=_MESH,
        scratch_types=[
            pltpu.VMEM((c,), jnp.int32),
            pltpu.VMEM((c,), jnp.int32),
            pltpu.VMEM((c, H), jnp.float32),
            pltpu.VMEM((c, H), jnp.float32),
            pltpu.SemaphoreType.DMA,
            pltpu.SemaphoreType.DMA,
        ],
    )
    def g1(tab_hbm, idx_hbm, out_hbm, idx0, idx1, r0, r1, s0, s1):
        cid = lax.axis_index("c")
        sid = lax.axis_index("s")
        wid = sid * NC + cid
        base0 = wid * n_w
        pltpu.sync_copy(idx_hbm.at[pl.ds(base0, c)], idx0)
        pltpu.async_copy(tab_hbm.at[idx0], r0, s0)

        def chunk(i2, carry):
            i = i2 * 2
            pltpu.sync_copy(idx_hbm.at[pl.ds(base0 + (i + 1) * c, c)], idx1)
            pltpu.async_copy(tab_hbm.at[idx1], r1, s1)
            pltpu.make_async_copy(tab_hbm.at[idx0], r0, s0).wait()
            pltpu.sync_copy(r0, out_hbm.at[pl.ds(base0 + i * c, c)])

            @pl.when(i2 < nch // 2 - 1)
            def _pref():
                pltpu.sync_copy(idx_hbm.at[pl.ds(base0 + (i + 2) * c, c)],
                                idx0)
                pltpu.async_copy(tab_hbm.at[idx0], r0, s0)

            pltpu.make_async_copy(tab_hbm.at[idx1], r1, s1).wait()
            pltpu.sync_copy(r1, out_hbm.at[pl.ds(base0 + (i + 1) * c, c)])
            return carry

        lax.fori_loop(0, nch // 2, chunk, 0)

    return g1


_gather_x = _make_gather1(S * B, 80)


_BPT = B // NW


@functools.partial(
    pl.kernel,
    out_type=(jax.ShapeDtypeStruct((B, H), jnp.float32),
              jax.ShapeDtypeStruct((B, H), jnp.float32)),
    mesh=_MESH,
    scratch_types=[
        pltpu.VMEM((_BPT,), jnp.int32),
        pltpu.VMEM((_BPT, H), jnp.float32),
        pltpu.VMEM((_BPT, H), jnp.float32),
        pltpu.VMEM((_BPT, H), jnp.float32),
        pltpu.SemaphoreType.DMA,
        pltpu.SemaphoreType.DMA,
        pltpu.SemaphoreType.DMA,
    ],
)
def _gather_user(u0_hbm, u1_hbm, uemb_hbm, idx_hbm, up_hbm, pu_hbm,
                 idx_v, ra, rb, rc, s0, s1, s2):
    cid = lax.axis_index("c")
    sid = lax.axis_index("s")
    wid = sid * NC + cid
    base = wid * _BPT
    pltpu.sync_copy(idx_hbm.at[pl.ds(base, _BPT)], idx_v)
    cp0 = pltpu.async_copy(u0_hbm.at[idx_v], ra, s0)
    cp1 = pltpu.async_copy(u1_hbm.at[idx_v], rb, s1)
    cp2 = pltpu.async_copy(uemb_hbm.at[idx_v], rc, s2)
    cp0.wait()
    cp1.wait()
    cp2.wait()

    def rowadd(rr, c2):
        for cc in range(H // LANES):
            sl = (rr, pl.ds(cc * LANES, LANES))
            ra[sl] = ra[sl] + rb[sl]
        return c2

    lax.fori_loop(0, _BPT, rowadd, 0)
    pltpu.sync_copy(ra, up_hbm.at[pl.ds(base, _BPT)])
    pltpu.sync_copy(rc, pu_hbm.at[pl.ds(base, _BPT)])


_OMEGA = float(2.0 * np.pi / 86400.0)
_KS = 5


def _c1_body(xe_ref, up_ref, pu_ref, t_ref, sx_ref, sy_ref, len_ref,
             h0_ref, wih_ref, whh_ref, bih_ref, bhh_ref, out_ref,
             h_s, acc_o, acc_w, tl, sxl, syl):
    pid = pl.program_id(0)

    @pl.when(pid == 0)
    def _init():
        h_s[...] = h0_ref[...]
        acc_o[...] = jnp.zeros_like(acc_o)
        acc_w[...] = jnp.zeros_like(acc_w)
        lm1 = len_ref[...] - 1
        ii = lax.broadcasted_iota(jnp.int32, (B, S), 1)
        selm = ii == lm1
        tl[...] = jnp.sum(jnp.where(selm, t_ref[...], 0.0), axis=1,
                          keepdims=True)
        sxl[...] = jnp.sum(jnp.where(selm, sx_ref[...], 0.0), axis=1,
                           keepdims=True)
        syl[...] = jnp.sum(jnp.where(selm, sy_ref[...], 0.0), axis=1,
                           keepdims=True)

    ii = lax.broadcasted_iota(jnp.int32, (B, S), 1)
    for k in range(_KS):
        i = pid * _KS + k
        xe = xe_ref[k]
        d = up_ref[...] - xe
        sim = jnp.exp(-jnp.sqrt(jnp.sum(d * d, axis=1, keepdims=True)
                                + 1e-12))

        gx = lax.dot_general(xe.astype(jnp.bfloat16), wih_ref[...],
                             (((1,), (1,)), ((), ())),
                             preferred_element_type=jnp.float32) + bih_ref[...]
        gh = lax.dot_general(h_s[...].astype(jnp.bfloat16), whh_ref[...],
                             (((1,), (1,)), ((), ())),
                             preferred_element_type=jnp.float32) + bhh_ref[...]
        xr, xz, xn = gx[:, :H], gx[:, H:2 * H], gx[:, 2 * H:]
        hr, hz, hn = gh[:, :H], gh[:, H:2 * H], gh[:, 2 * H:]
        r = jax.nn.sigmoid(xr + hr)
        z = jax.nn.sigmoid(xz + hz)
        nn = jnp.tanh(xn + r * hn)
        h_new = (1.0 - z) * nn + z * h_s[...]
        valid = i < len_ref[...]
        h_s[...] = jnp.where(valid, h_new, h_s[...])
        o = jnp.where(valid, h_new, 0.0)

        sel = ii == i
        t_i = jnp.sum(jnp.where(sel, t_ref[...], 0.0), axis=1, keepdims=True)
        sx_i = jnp.sum(jnp.where(sel, sx_ref[...], 0.0), axis=1,
                       keepdims=True)
        sy_i = jnp.sum(jnp.where(sel, sy_ref[...], 0.0), axis=1,
                       keepdims=True)

        dt = tl[...] - t_i
        a = (jnp.cos(dt * _OMEGA) + 1.0) * 0.5 * jnp.exp(dt * (-1e-5))
        dsx = sxl[...] - sx_i
        dsy = syl[...] - sy_i
        bw = jnp.exp(-jnp.sqrt(dsx * dsx + dsy * dsy + 1e-12))
        w = a * bw * sim
        w = jnp.where(valid, w, 0.0)
        acc_o[...] = acc_o[...] + w * o
        acc_w[...] = acc_w[...] + w

    @pl.when(pid == S // _KS - 1)
    def _fin():
        out_ref[:, :H] = acc_o[...] / acc_w[...]
        out_ref[:, H:] = pu_ref[...]


def _c1(x_emb3, up, pu, t_bt, sx_bt, sy_bt, len_b1, h0,
        W_ih, W_hh, b_ih2, b_hh2):
    full = lambda shape: pl.BlockSpec(shape, lambda i: tuple(0 for _ in shape))
    return pl.pallas_call(
        _c1_body,
        grid=(S // _KS,),
        in_specs=[
            pl.BlockSpec((_KS, B, H), lambda i: (i, 0, 0)),
            full((B, H)), full((B, H)),
            full((B, S)), full((B, S)), full((B, S)),
            full((B, 1)), full((B, H)),
            full((3 * H, H)), full((3 * H, H)),
            full((1, 3 * H)), full((1, 3 * H)),
        ],
        out_specs=full((B, 2 * H)),
        out_shape=jax.ShapeDtypeStruct((B, 2 * H), jnp.float32),
        scratch_shapes=[
            pltpu.VMEM((B, H), jnp.float32),
            pltpu.VMEM((B, H), jnp.float32),
            pltpu.VMEM((B, 1), jnp.float32),
            pltpu.VMEM((B, 1), jnp.float32),
            pltpu.VMEM((B, 1), jnp.float32),
            pltpu.VMEM((B, 1), jnp.float32),
        ],
    )(x_emb3, up, pu, t_bt, sx_bt, sy_bt, len_b1, h0, W_ih, W_hh,
      b_ih2, b_hh2)


_CP = 2048


def _c2_body(pu_ref, w_ref, b_ref, out_ref):
    out_ref[...] = lax.dot_general(
        pu_ref[...], w_ref[...], (((1,), (1,)), ((), ())),
        preferred_element_type=jnp.float32) + b_ref[...]


def _c2(out_pu, fc_W, fc_b2):
    return pl.pallas_call(
        _c2_body,
        grid=((P + _CP - 1) // _CP,),
        in_specs=[
            pl.BlockSpec((B, 2 * H), lambda i: (0, 0)),
            pl.BlockSpec((_CP, 2 * H), lambda i: (i, 0)),
            pl.BlockSpec((1, _CP), lambda i: (0, i)),
        ],
        out_specs=pl.BlockSpec((B, _CP), lambda i: (0, i)),
        out_shape=jax.ShapeDtypeStruct((B, P), jnp.float32),
    )(out_pu, fc_W, fc_b2)


def kernel(x, t, s, y_t, y_s, h, active_user, length, emb, user_emb,
           W_ih, W_hh, b_ih, b_hh, fc_W, fc_b,
           trans_row, trans_col, trans_val, inter_row, inter_col, inter_val):
    x_flat = x.reshape(-1).astype(jnp.int32)
    au = active_user.reshape(-1).astype(jnp.int32)

    tp = _spmm(trans_row.astype(jnp.int32), trans_col.astype(jnp.int32),
               trans_val, emb)
    upar = _spmm(inter_row.astype(jnp.int32), inter_col.astype(jnp.int32),
                 inter_val, emb)

    x_emb = _gather_x(_addtab(tp), x_flat)
    up, pu = _gather_user(upar[0], upar[1], user_emb, au)

    out_pu = _c1(x_emb.reshape(S, B, H), up, pu,
                 t.T, s[..., 0].T, s[..., 1].T,
                 length.reshape(B, 1).astype(jnp.int32), h[0],
                 W_ih.astype(jnp.bfloat16), W_hh.astype(jnp.bfloat16),
                 b_ih.reshape(1, -1), b_hh.reshape(1, -1))
    return _c2(out_pu, fc_W, fc_b.reshape(1, -1))
